# baseline (speedup 1.0000x reference)
import jax
import jax.numpy as jnp
from jax import lax
from jax.experimental import pallas as pl
from jax.experimental.pallas import tpu as pltpu
from jax.experimental.pallas import tpu_sc as plsc

NUM_FACTORS = 32
BATCH = 16384

NC = 2
NS = 16
NW = NC * NS
B_PER_W = BATCH // NW
CHUNK = 128
NCHUNK = B_PER_W // CHUNK


def _sc_gather(uid2, iid2, user_embeddings, user_bias, item_embeddings,
               item_bias):
    mesh = plsc.VectorSubcoreMesh(core_axis_name="c", subcore_axis_name="s")
    f32 = jnp.float32

    kern = pl.kernel(
        _sc_gather_body,
        out_type=(
            jax.ShapeDtypeStruct((BATCH, NUM_FACTORS), f32),
            jax.ShapeDtypeStruct((BATCH, NUM_FACTORS), f32),
            jax.ShapeDtypeStruct((BATCH, 1), f32),
            jax.ShapeDtypeStruct((BATCH, 1), f32),
        ),
        mesh=mesh,
        scratch_types=[
            pltpu.VMEM((NCHUNK, CHUNK), jnp.int32),
            pltpu.VMEM((NCHUNK, CHUNK), jnp.int32),
            pltpu.VMEM((B_PER_W, NUM_FACTORS), f32),
            pltpu.VMEM((B_PER_W, NUM_FACTORS), f32),
            pltpu.VMEM((B_PER_W, 1), f32),
            pltpu.VMEM((B_PER_W, 1), f32),
            pltpu.SemaphoreType.DMA,
        ],
        compiler_params=pltpu.CompilerParams(use_tc_tiling_on_sc=False),
    )
    return kern(uid2, iid2, user_embeddings, user_bias, item_embeddings,
                item_bias)


def _sc_gather_body(uid_hbm, iid_hbm, ue_hbm, ub_hbm, ie_hbm, ib_hbm,
                    p_hbm, q_hbm, bu_hbm, bi_hbm,
                    idx_u, idx_i, pu_v, qi_v, bu_v, bi_v, sem):
    wid = lax.axis_index("s") * NC + lax.axis_index("c")
    row0 = wid * NCHUNK
    pltpu.sync_copy(uid_hbm.at[pl.ds(row0, NCHUNK)], idx_u)
    pltpu.sync_copy(iid_hbm.at[pl.ds(row0, NCHUNK)], idx_i)

    copies = []
    for ch in range(NCHUNK):
        dst = pl.ds(ch * CHUNK, CHUNK)
        copies.append(pltpu.async_copy(ue_hbm.at[idx_u.at[ch]],
                                       pu_v.at[dst], sem))
        copies.append(pltpu.async_copy(ie_hbm.at[idx_i.at[ch]],
                                       qi_v.at[dst], sem))
        copies.append(pltpu.async_copy(ub_hbm.at[idx_u.at[ch]],
                                       bu_v.at[dst], sem))
        copies.append(pltpu.async_copy(ib_hbm.at[idx_i.at[ch]],
                                       bi_v.at[dst], sem))
    for cp in copies:
        cp.wait()

    base = wid * B_PER_W
    out = pl.ds(base, B_PER_W)
    pltpu.sync_copy(pu_v, p_hbm.at[out])
    pltpu.sync_copy(qi_v, q_hbm.at[out])
    pltpu.sync_copy(bu_v, bu_hbm.at[out])
    pltpu.sync_copy(bi_v, bi_hbm.at[out])


def _tc_reduce_body(p_ref, q_ref, bu_ref, bi_ref, o_ref):
    prod = p_ref[...] * q_ref[...]
    o_ref[...] = (jnp.sum(prod, axis=1) + bu_ref[...][:, 0]
                  + bi_ref[...][:, 0])


def _tc_reduce(P, Q, BU, BI):
    blk = 2048
    grid = (BATCH // blk,)
    return pl.pallas_call(
        _tc_reduce_body,
        grid=grid,
        in_specs=[
            pl.BlockSpec((blk, NUM_FACTORS), lambda i: (i, 0)),
            pl.BlockSpec((blk, NUM_FACTORS), lambda i: (i, 0)),
            pl.BlockSpec((blk, 1), lambda i: (i, 0)),
            pl.BlockSpec((blk, 1), lambda i: (i, 0)),
        ],
        out_specs=pl.BlockSpec((blk,), lambda i: (i,)),
        out_shape=jax.ShapeDtypeStruct((BATCH,), jnp.float32),
    )(P, Q, BU, BI)


def kernel(user_id, item_id, user_embeddings, user_bias, item_embeddings,
           item_bias):
    uid2 = user_id.astype(jnp.int32).reshape(BATCH // CHUNK, CHUNK)
    iid2 = item_id.astype(jnp.int32).reshape(BATCH // CHUNK, CHUNK)
    P, Q, BU, BI = _sc_gather(uid2, iid2, user_embeddings, user_bias,
                              item_embeddings, item_bias)
    return _tc_reduce(P, Q, BU, BI)


# speedup vs baseline: 1.0110x; 1.0110x over previous
import jax
import jax.numpy as jnp
from jax import lax
from jax.experimental import pallas as pl
from jax.experimental.pallas import tpu as pltpu
from jax.experimental.pallas import tpu_sc as plsc

NUM_FACTORS = 32
BATCH = 16384

NC = 2
NS = 16
NW = NC * NS
B_PER_W = BATCH // NW
VLEN = 16
CHUNK = 128
NCHUNK = B_PER_W // CHUNK

EMB_STRIDE = 128 // NUM_FACTORS
BIAS_STRIDE = 128


def _sc_gather(uid2, iid2, user_embeddings, user_bias, item_embeddings,
               item_bias):
    mesh = plsc.VectorSubcoreMesh(core_axis_name="c", subcore_axis_name="s")
    f32 = jnp.float32

    kern = pl.kernel(
        _sc_gather_body,
        out_type=(
            jax.ShapeDtypeStruct((BATCH * NUM_FACTORS // 128, 128), f32),
            jax.ShapeDtypeStruct((BATCH // 128, 128), f32),
        ),
        mesh=mesh,
        scratch_types=[
            pltpu.VMEM((NCHUNK, CHUNK), jnp.int32),
            pltpu.VMEM((NCHUNK, CHUNK), jnp.int32),
            pltpu.VMEM((NCHUNK, CHUNK), jnp.int32),
            pltpu.VMEM((NCHUNK, CHUNK), jnp.int32),
            pltpu.VMEM((NCHUNK, CHUNK), jnp.int32),
            pltpu.VMEM((NCHUNK, CHUNK), jnp.int32),
            pltpu.VMEM((B_PER_W, NUM_FACTORS), f32),
            pltpu.VMEM((B_PER_W, NUM_FACTORS), f32),
            pltpu.VMEM((B_PER_W, 1), f32),
            pltpu.VMEM((B_PER_W, 1), f32),
            pltpu.VMEM((NCHUNK, CHUNK), f32),
            pltpu.VMEM((B_PER_W * NUM_FACTORS // 128, 128), f32),
            pltpu.SemaphoreType.DMA,
        ],
        compiler_params=pltpu.CompilerParams(
            use_tc_tiling_on_sc=False,
            disable_bounds_checks=True,
            needs_layout_passes=False,
        ),
    )
    return kern(uid2, iid2, user_embeddings, user_bias, item_embeddings,
                item_bias)


def _sc_gather_body(uid_hbm, iid_hbm, ue_hbm, ub_hbm, ie_hbm, ib_hbm,
                    prod_hbm, bsum_hbm,
                    idx_u, idx_i, idx_ue, idx_ie, idx_ub, idx_ib,
                    pu_v, qi_v, bu_v, bi_v, bs_v, pw_v, sem):
    wid = lax.axis_index("s") * NC + lax.axis_index("c")
    row0 = wid * NCHUNK
    pltpu.sync_copy(uid_hbm.at[pl.ds(row0, NCHUNK)], idx_u)
    pltpu.sync_copy(iid_hbm.at[pl.ds(row0, NCHUNK)], idx_i)

    for ch in range(NCHUNK):
        for k in range(CHUNK // VLEN):
            sl = pl.ds(k * VLEN, VLEN)
            u = idx_u[ch, sl]
            i = idx_i[ch, sl]
            idx_ue[ch, sl] = u * EMB_STRIDE
            idx_ie[ch, sl] = i * EMB_STRIDE
            idx_ub[ch, sl] = u * BIAS_STRIDE
            idx_ib[ch, sl] = i * BIAS_STRIDE

    copies = []
    for ch in range(NCHUNK):
        dst = pl.ds(ch * CHUNK, CHUNK)
        copies.append(pltpu.async_copy(ue_hbm.at[idx_ue.at[ch]],
                                       pu_v.at[dst], sem))
        copies.append(pltpu.async_copy(ie_hbm.at[idx_ie.at[ch]],
                                       qi_v.at[dst], sem))
        copies.append(pltpu.async_copy(ub_hbm.at[idx_ub.at[ch]],
                                       bu_v.at[dst], sem))
        copies.append(pltpu.async_copy(ib_hbm.at[idx_ib.at[ch]],
                                       bi_v.at[dst], sem))
    for cp in copies:
        cp.wait()

    zero16 = jnp.zeros((VLEN,), jnp.int32)
    iota16 = lax.iota(jnp.int32, VLEN)
    for ch in range(NCHUNK):
        for k in range(CHUNK // VLEN):
            rows = iota16 + (ch * CHUNK + k * VLEN)
            vu = plsc.load_gather(bu_v, [rows, zero16])
            vi = plsc.load_gather(bi_v, [rows, zero16])
            bs_v[ch, pl.ds(k * VLEN, VLEN)] = vu + vi

    @pl.loop(0, B_PER_W * NUM_FACTORS // 128)
    def _(r):
        for k in range(128 // VLEN):
            row = 4 * r + k // 2
            col = pl.ds((k % 2) * VLEN, VLEN)
            pw_v[r, pl.ds(k * VLEN, VLEN)] = pu_v[row, col] * qi_v[row, col]

    nrow = B_PER_W * NUM_FACTORS // 128
    pltpu.sync_copy(pw_v, prod_hbm.at[pl.ds(wid * nrow, nrow)])
    pltpu.sync_copy(bs_v, bsum_hbm.at[pl.ds(wid * NCHUNK, NCHUNK)])


def _tc_reduce_body(p_ref, bs_ref, o_ref):
    blk = o_ref.shape[0]
    prod = p_ref[...]
    sums = jnp.sum(prod.reshape(blk // 4, 4, NUM_FACTORS), axis=-1)
    o_ref[...] = sums.reshape(blk) + bs_ref[...].reshape(blk)


def _tc_reduce(PROD, BSUM):
    blk = 2048
    rows = blk * NUM_FACTORS // 128
    brows = blk // 128
    return pl.pallas_call(
        _tc_reduce_body,
        grid=(BATCH // blk,),
        in_specs=[
            pl.BlockSpec((rows, 128), lambda i: (i, 0)),
            pl.BlockSpec((brows, 128), lambda i: (i, 0)),
        ],
        out_specs=pl.BlockSpec((blk,), lambda i: (i,)),
        out_shape=jax.ShapeDtypeStruct((BATCH,), jnp.float32),
    )(PROD, BSUM)


def kernel(user_id, item_id, user_embeddings, user_bias, item_embeddings,
           item_bias):
    uid2 = user_id.astype(jnp.int32).reshape(BATCH // 128, 128)
    iid2 = item_id.astype(jnp.int32).reshape(BATCH // 128, 128)
    PROD, BSUM = _sc_gather(uid2, iid2, user_embeddings, user_bias,
                            item_embeddings, item_bias)
    return _tc_reduce(PROD, BSUM)


# speedup vs baseline: 2.4804x; 2.4535x over previous
import jax
import jax.numpy as jnp
from jax import lax
from jax.experimental import pallas as pl
from jax.experimental.pallas import tpu as pltpu
from jax.experimental.pallas import tpu_sc as plsc

NUM_FACTORS = 32
BATCH = 16384

NC = 2
NS = 16
NW = NC * NS
B_PER_W = BATCH // NW
VLEN = 16
CHUNK = 128
NCHUNK = B_PER_W // CHUNK
PROD_ROWS = B_PER_W * NUM_FACTORS // 128


def _sc_gather(uid2, iid2, user_embeddings, user_bias, item_embeddings,
               item_bias):
    mesh = plsc.VectorSubcoreMesh(core_axis_name="c", subcore_axis_name="s")
    f32 = jnp.float32

    kern = pl.kernel(
        _sc_gather_body,
        out_type=(
            jax.ShapeDtypeStruct((BATCH * NUM_FACTORS // 128, 128), f32),
            jax.ShapeDtypeStruct((BATCH // 128, 128), f32),
        ),
        mesh=mesh,
        scratch_types=[
            pltpu.VMEM((NCHUNK, CHUNK), jnp.int32),
            pltpu.VMEM((NCHUNK, CHUNK), jnp.int32),
            pltpu.VMEM((CHUNK, NUM_FACTORS), f32),
            pltpu.VMEM((CHUNK, NUM_FACTORS), f32),
            pltpu.VMEM((CHUNK, 1), f32),
            pltpu.VMEM((CHUNK, 1), f32),
            pltpu.VMEM((NCHUNK, CHUNK), f32),
            pltpu.VMEM((PROD_ROWS, 128), f32),
            pltpu.SemaphoreType.DMA,
        ],
        compiler_params=pltpu.CompilerParams(
            disable_bounds_checks=True,
            needs_layout_passes=False,
        ),
    )
    return kern(uid2, iid2, user_embeddings, user_bias, item_embeddings,
                item_bias)


def _sc_gather_body(uid_hbm, iid_hbm, ue_hbm, ub_hbm, ie_hbm, ib_hbm,
                    prod_hbm, bsum_hbm,
                    idx_u, idx_i, pu_c, qi_c, bu_c, bi_c, bs_v, pw_v, sem):
    wid = lax.axis_index("s") * NC + lax.axis_index("c")
    row0 = wid * NCHUNK
    pltpu.sync_copy(uid_hbm.at[pl.ds(row0, NCHUNK)], idx_u)
    pltpu.sync_copy(iid_hbm.at[pl.ds(row0, NCHUNK)], idx_i)

    zero16 = jnp.zeros((VLEN,), jnp.int32)
    iota16 = lax.iota(jnp.int32, VLEN)

    for ch in range(NCHUNK):
        @pl.loop(0, CHUNK // VLEN)
        def _(g):
            uvec = idx_u[ch, pl.ds(g * VLEN, VLEN)]
            ivec = idx_i[ch, pl.ds(g * VLEN, VLEN)]
            for j in range(VLEN):
                u = uvec[j]
                i = ivec[j]
                dst = pl.ds(g * VLEN + j, 1)
                pltpu.async_copy(ue_hbm.at[pl.ds(u, 1)], pu_c.at[dst], sem)
                pltpu.async_copy(ie_hbm.at[pl.ds(i, 1)], qi_c.at[dst], sem)
                pltpu.async_copy(ub_hbm.at[pl.ds(u, 1)], bu_c.at[dst], sem)
                pltpu.async_copy(ib_hbm.at[pl.ds(i, 1)], bi_c.at[dst], sem)

        @pl.loop(0, CHUNK)
        def _(k):
            dst = pl.ds(k, 1)
            pltpu.make_async_copy(ue_hbm.at[pl.ds(0, 1)],
                                  pu_c.at[dst], sem).wait()
            pltpu.make_async_copy(ie_hbm.at[pl.ds(0, 1)],
                                  qi_c.at[dst], sem).wait()
            pltpu.make_async_copy(ub_hbm.at[pl.ds(0, 1)],
                                  bu_c.at[dst], sem).wait()
            pltpu.make_async_copy(ib_hbm.at[pl.ds(0, 1)],
                                  bi_c.at[dst], sem).wait()

        @pl.loop(0, CHUNK // 4)
        def _(r):
            for k2 in range(128 // VLEN):
                row = 4 * r + k2 // 2
                col = pl.ds((k2 % 2) * VLEN, VLEN)
                pw_v[ch * (CHUNK // 4) + r, pl.ds(k2 * VLEN, VLEN)] = (
                    pu_c[row, col] * qi_c[row, col])

        for k2 in range(CHUNK // VLEN):
            rows16 = iota16 + k2 * VLEN
            vu = plsc.load_gather(bu_c, [rows16, zero16])
            vi = plsc.load_gather(bi_c, [rows16, zero16])
            bs_v[ch, pl.ds(k2 * VLEN, VLEN)] = vu + vi

    pltpu.sync_copy(pw_v, prod_hbm.at[pl.ds(wid * PROD_ROWS, PROD_ROWS)])
    pltpu.sync_copy(bs_v, bsum_hbm.at[pl.ds(wid * NCHUNK, NCHUNK)])


def _tc_reduce_body(p_ref, bs_ref, o_ref):
    blk = o_ref.shape[0]
    prod = p_ref[...]
    sums = jnp.sum(prod.reshape(blk // 4, 4, NUM_FACTORS), axis=-1)
    o_ref[...] = sums.reshape(blk) + bs_ref[...].reshape(blk)


def _tc_reduce(PROD, BSUM):
    blk = 2048
    rows = blk * NUM_FACTORS // 128
    brows = blk // 128
    return pl.pallas_call(
        _tc_reduce_body,
        grid=(BATCH // blk,),
        in_specs=[
            pl.BlockSpec((rows, 128), lambda i: (i, 0)),
            pl.BlockSpec((brows, 128), lambda i: (i, 0)),
        ],
        out_specs=pl.BlockSpec((blk,), lambda i: (i,)),
        out_shape=jax.ShapeDtypeStruct((BATCH,), jnp.float32),
    )(PROD, BSUM)


def kernel(user_id, item_id, user_embeddings, user_bias, item_embeddings,
           item_bias):
    uid2 = user_id.astype(jnp.int32).reshape(BATCH // 128, 128)
    iid2 = item_id.astype(jnp.int32).reshape(BATCH // 128, 128)
    PROD, BSUM = _sc_gather(uid2, iid2, user_embeddings, user_bias,
                            item_embeddings, item_bias)
    return _tc_reduce(PROD, BSUM)
